# baseline (speedup 1.0000x reference)
import numpy as np

import jax
import jax.numpy as jnp
from jax.experimental import pallas as pl
from jax.experimental.pallas import tpu as pltpu

N = 1024
R = 4
IN_DIM = 256
HID = 256
D = IN_DIM * R
BF = jnp.bfloat16
F32 = jnp.float32

_a = np.arange(D)
_S3_np = np.zeros((R, D, IN_DIM), dtype=np.float32)
for _r in range(R):
    _S3_np[_r, _a[_a % R == _r], (_a[_a % R == _r] // R)] = 1.0
_ST_np = _S3_np.transpose(0, 2, 1).reshape(D, D)


def _body(adj_ref, xf_ref, hf_ref, s3_ref, st_ref,
          wxz_ref, wxr_ref, wxh_ref, whz_ref, whr_ref, out_ref, hn_s):
    xf16 = xf_ref[...].astype(BF)
    hf16 = hf_ref[...].astype(BF)
    for r in range(R):
        s3 = s3_ref[r]
        xd = jnp.dot(xf16, s3, preferred_element_type=F32).astype(BF)
        h32 = jnp.dot(hf16, s3, preferred_element_type=F32)
        hd = h32.astype(BF)
        a16 = adj_ref[r].astype(BF)
        ax = jnp.dot(a16, xd, preferred_element_type=F32).astype(BF)
        ah = jnp.dot(a16, hd, preferred_element_type=F32).astype(BF)
        wxz = wxz_ref[r].astype(BF)
        wxr = wxr_ref[r].astype(BF)
        wxh = wxh_ref[r].astype(BF)
        whz = whz_ref[r].astype(BF)
        whr = whr_ref[r].astype(BF)
        zpre = (jnp.dot(ax, wxz, preferred_element_type=F32)
                + jnp.dot(ah, whz, preferred_element_type=F32))
        rpre = (jnp.dot(ax, wxr, preferred_element_type=F32)
                + jnp.dot(ah, whr, preferred_element_type=F32))
        z = jax.nn.sigmoid(zpre)
        rg = jax.nn.sigmoid(rpre)
        tterm = jnp.dot(ax, wxh, preferred_element_type=F32)
        g16 = (rg * h32).astype(BF)
        ag = jnp.dot(a16, g16, preferred_element_type=F32).astype(BF)
        ht = jnp.tanh(tterm + jnp.dot(ag, whr, preferred_element_type=F32))
        hn = z * h32 + (1.0 - z) * ht
        hn_s[:, r * HID:(r + 1) * HID] = hn.astype(BF)
    out_ref[...] = jnp.dot(hn_s[...], st_ref[...],
                           preferred_element_type=F32)


def kernel(X, adj, h_pre, W_xz, W_xr, W_xh, W_hz, W_hr, W_hh):
    del W_hh
    Xf = X.reshape(N, D)
    Hf = h_pre.reshape(N, D)
    S3 = jnp.asarray(_S3_np, dtype=BF)
    ST = jnp.asarray(_ST_np, dtype=BF)

    def full(*shape):
        return pl.BlockSpec(shape, lambda: tuple(0 for _ in shape))

    out = pl.pallas_call(
        _body,
        grid=(),
        in_specs=[
            full(R, N, N),
            full(N, D),
            full(N, D),
            full(R, D, IN_DIM),
            full(D, D),
            full(R, IN_DIM, HID),
            full(R, IN_DIM, HID),
            full(R, IN_DIM, HID),
            full(R, HID, HID),
            full(R, HID, HID),
        ],
        out_specs=full(N, D),
        out_shape=jax.ShapeDtypeStruct((N, D), F32),
        scratch_shapes=[pltpu.VMEM((N, D), BF)],
    )(adj, Xf, Hf, S3, ST, W_xz, W_xr, W_xh, W_hz, W_hr)

    return out.reshape(N, HID, R)


# speedup vs baseline: 1.1865x; 1.1865x over previous
import jax
import jax.numpy as jnp
from jax.experimental import pallas as pl
from jax.experimental.pallas import tpu as pltpu

N = 1024
R = 4
IN_DIM = 256
HID = 256
BN = 256
NB = N // BN
BF = jnp.bfloat16
F32 = jnp.float32


def _body(adj_ref, xh_ref, w1_ref, w2_ref, h_ref, out_ref, z_s, t_s, g_s):
    p = pl.program_id(1)
    i = pl.program_id(2)
    a16 = adj_ref[0].astype(BF)

    @pl.when(p == 0)
    def _phase0():
        axh = jnp.dot(a16, xh_ref[0].astype(BF), preferred_element_type=F32)
        pre = jnp.dot(axh.astype(BF), w1_ref[0].astype(BF),
                      preferred_element_type=F32)
        z = jax.nn.sigmoid(pre[:, :HID])
        rg = jax.nn.sigmoid(pre[:, HID:2 * HID])
        z_s[pl.ds(i * BN, BN), :] = z
        t_s[pl.ds(i * BN, BN), :] = pre[:, 2 * HID:]
        g_s[pl.ds(i * BN, BN), :] = (rg * h_ref[0]).astype(BF)
        out_ref[0, 0] = jnp.zeros((BN, HID), F32)

    @pl.when(p == 1)
    def _phase1():
        ag = jnp.dot(a16, g_s[...], preferred_element_type=F32)
        ht = jnp.tanh(t_s[pl.ds(i * BN, BN), :]
                      + jnp.dot(ag.astype(BF), w2_ref[0].astype(BF),
                                preferred_element_type=F32))
        z = z_s[pl.ds(i * BN, BN), :]
        out_ref[0, 0] = z * h_ref[0] + (1.0 - z) * ht


def kernel(X, adj, h_pre, W_xz, W_xr, W_xh, W_hz, W_hr, W_hh):
    del W_hh
    Xr = jnp.transpose(X, (2, 0, 1))
    Hr = jnp.transpose(h_pre, (2, 0, 1))
    XH = jnp.concatenate([Xr, Hr], axis=2)
    W_top = jnp.concatenate([W_xz, W_xr, W_xh], axis=2)
    W_bot = jnp.concatenate([W_hz, W_hr, jnp.zeros_like(W_hr)], axis=2)
    W1 = jnp.concatenate([W_top, W_bot], axis=1)

    out = pl.pallas_call(
        _body,
        grid=(R, 2, NB),
        in_specs=[
            pl.BlockSpec((1, BN, N), lambda r, p, i: (r, i, 0)),
            pl.BlockSpec((1, N, IN_DIM + HID), lambda r, p, i: (r, 0, 0)),
            pl.BlockSpec((1, IN_DIM + HID, 3 * HID),
                         lambda r, p, i: (r, 0, 0)),
            pl.BlockSpec((1, HID, HID), lambda r, p, i: (r, 0, 0)),
            pl.BlockSpec((1, BN, HID), lambda r, p, i: (r, i, 0)),
        ],
        out_specs=pl.BlockSpec((1, 1, BN, HID), lambda r, p, i: (p, r, i, 0)),
        out_shape=jax.ShapeDtypeStruct((2, R, N, HID), F32),
        scratch_shapes=[
            pltpu.VMEM((N, HID), F32),
            pltpu.VMEM((N, HID), F32),
            pltpu.VMEM((N, HID), BF),
        ],
        compiler_params=pltpu.CompilerParams(
            dimension_semantics=("arbitrary", "arbitrary", "arbitrary"),
        ),
    )(adj, XH, W1, W_hr, Hr)

    return jnp.transpose(out[1], (1, 2, 0))


# speedup vs baseline: 1.4287x; 1.2041x over previous
import jax
import jax.numpy as jnp
from jax.experimental import pallas as pl
from jax.experimental.pallas import tpu as pltpu

N = 1024
R = 4
IN_DIM = 256
HID = 256
BN = 512
NB = N // BN
BF = jnp.bfloat16
F32 = jnp.float32


def _body(adj_ref, xh_ref, w1_ref, w2_ref, h_ref, out_ref, z_s, t_s, g_s):
    p = pl.program_id(1)
    i = pl.program_id(2)
    a16 = adj_ref[0].astype(BF)

    @pl.when(p == 0)
    def _phase0():
        axh = jnp.dot(a16, xh_ref[0].astype(BF), preferred_element_type=F32)
        pre = jnp.dot(axh.astype(BF), w1_ref[0].astype(BF),
                      preferred_element_type=F32)
        z = jax.nn.sigmoid(pre[:, :HID])
        rg = jax.nn.sigmoid(pre[:, HID:2 * HID])
        z_s[pl.ds(i * BN, BN), :] = z
        t_s[pl.ds(i * BN, BN), :] = pre[:, 2 * HID:]
        g_s[pl.ds(i * BN, BN), :] = (rg * h_ref[0]).astype(BF)
        out_ref[0, 0] = jnp.zeros((BN, HID), F32)

    @pl.when(p == 1)
    def _phase1():
        ag = jnp.dot(a16, g_s[...], preferred_element_type=F32)
        ht = jnp.tanh(t_s[pl.ds(i * BN, BN), :]
                      + jnp.dot(ag.astype(BF), w2_ref[0].astype(BF),
                                preferred_element_type=F32))
        z = z_s[pl.ds(i * BN, BN), :]
        out_ref[0, 0] = z * h_ref[0] + (1.0 - z) * ht


def kernel(X, adj, h_pre, W_xz, W_xr, W_xh, W_hz, W_hr, W_hh):
    del W_hh
    Xr = jnp.transpose(X, (2, 0, 1))
    Hr = jnp.transpose(h_pre, (2, 0, 1))
    XH = jnp.concatenate([Xr, Hr], axis=2)
    W_top = jnp.concatenate([W_xz, W_xr, W_xh], axis=2)
    W_bot = jnp.concatenate([W_hz, W_hr, jnp.zeros_like(W_hr)], axis=2)
    W1 = jnp.concatenate([W_top, W_bot], axis=1)

    out = pl.pallas_call(
        _body,
        grid=(R, 2, NB),
        in_specs=[
            pl.BlockSpec((1, BN, N), lambda r, p, i: (r, i, 0)),
            pl.BlockSpec((1, N, IN_DIM + HID), lambda r, p, i: (r, 0, 0)),
            pl.BlockSpec((1, IN_DIM + HID, 3 * HID),
                         lambda r, p, i: (r, 0, 0)),
            pl.BlockSpec((1, HID, HID), lambda r, p, i: (r, 0, 0)),
            pl.BlockSpec((1, BN, HID), lambda r, p, i: (r, i, 0)),
        ],
        out_specs=pl.BlockSpec((1, 1, BN, HID), lambda r, p, i: (p, r, i, 0)),
        out_shape=jax.ShapeDtypeStruct((2, R, N, HID), F32),
        scratch_shapes=[
            pltpu.VMEM((N, HID), F32),
            pltpu.VMEM((N, HID), F32),
            pltpu.VMEM((N, HID), BF),
        ],
        compiler_params=pltpu.CompilerParams(
            dimension_semantics=("arbitrary", "arbitrary", "arbitrary"),
        ),
    )(adj, XH, W1, W_hr, Hr)

    return jnp.transpose(out[1], (1, 2, 0))


# speedup vs baseline: 1.4954x; 1.0467x over previous
import jax
import jax.numpy as jnp
from jax.experimental import pallas as pl
from jax.experimental.pallas import tpu as pltpu

N = 1024
R = 4
IN_DIM = 256
HID = 256
BN = 1024
NB = N // BN
BF = jnp.bfloat16
F32 = jnp.float32


def _body(adj_ref, xh_ref, w1_ref, w2_ref, h_ref, out_ref, z_s, t_s, g_s):
    p = pl.program_id(1)
    i = pl.program_id(2)
    a16 = adj_ref[0].astype(BF)

    @pl.when(p == 0)
    def _phase0():
        axh = jnp.dot(a16, xh_ref[0].astype(BF), preferred_element_type=F32)
        pre = jnp.dot(axh.astype(BF), w1_ref[0].astype(BF),
                      preferred_element_type=F32)
        z = jax.nn.sigmoid(pre[:, :HID])
        rg = jax.nn.sigmoid(pre[:, HID:2 * HID])
        z_s[pl.ds(i * BN, BN), :] = z
        t_s[pl.ds(i * BN, BN), :] = pre[:, 2 * HID:]
        g_s[pl.ds(i * BN, BN), :] = (rg * h_ref[0]).astype(BF)
        out_ref[0, 0] = jnp.zeros((BN, HID), F32)

    @pl.when(p == 1)
    def _phase1():
        ag = jnp.dot(a16, g_s[...], preferred_element_type=F32)
        ht = jnp.tanh(t_s[pl.ds(i * BN, BN), :]
                      + jnp.dot(ag.astype(BF), w2_ref[0].astype(BF),
                                preferred_element_type=F32))
        z = z_s[pl.ds(i * BN, BN), :]
        out_ref[0, 0] = z * h_ref[0] + (1.0 - z) * ht


def kernel(X, adj, h_pre, W_xz, W_xr, W_xh, W_hz, W_hr, W_hh):
    del W_hh
    Xr = jnp.transpose(X, (2, 0, 1))
    Hr = jnp.transpose(h_pre, (2, 0, 1))
    XH = jnp.concatenate([Xr, Hr], axis=2)
    W_top = jnp.concatenate([W_xz, W_xr, W_xh], axis=2)
    W_bot = jnp.concatenate([W_hz, W_hr, jnp.zeros_like(W_hr)], axis=2)
    W1 = jnp.concatenate([W_top, W_bot], axis=1)

    out = pl.pallas_call(
        _body,
        grid=(R, 2, NB),
        in_specs=[
            pl.BlockSpec((1, BN, N), lambda r, p, i: (r, i, 0)),
            pl.BlockSpec((1, N, IN_DIM + HID), lambda r, p, i: (r, 0, 0)),
            pl.BlockSpec((1, IN_DIM + HID, 3 * HID),
                         lambda r, p, i: (r, 0, 0)),
            pl.BlockSpec((1, HID, HID), lambda r, p, i: (r, 0, 0)),
            pl.BlockSpec((1, BN, HID), lambda r, p, i: (r, i, 0)),
        ],
        out_specs=pl.BlockSpec((1, 1, BN, HID), lambda r, p, i: (p, r, i, 0)),
        out_shape=jax.ShapeDtypeStruct((2, R, N, HID), F32),
        scratch_shapes=[
            pltpu.VMEM((N, HID), F32),
            pltpu.VMEM((N, HID), F32),
            pltpu.VMEM((N, HID), BF),
        ],
        compiler_params=pltpu.CompilerParams(
            dimension_semantics=("arbitrary", "arbitrary", "arbitrary"),
        ),
    )(adj, XH, W1, W_hr, Hr)

    return jnp.transpose(out[1], (1, 2, 0))
